# Initial kernel scaffold; baseline (speedup 1.0000x reference)
import jax
import jax.numpy as jnp
from jax.experimental import pallas as pl


def kernel(x, edge_index, We1, be1, We2, be2, Wd1, bd1, Wd2, bd2):
    raise NotImplementedError("write your pallas kernel here")


# speedup vs baseline: 15.1692x; 15.1692x over previous
import functools

import jax
import jax.numpy as jnp
from jax import lax
from jax.experimental import pallas as pl
from jax.experimental.pallas import tpu as pltpu
from jax.experimental.pallas import tpu_sc as plsc

_C = 128
_NC = 2
_NS = 16


def _pad_rows(n):
    return (-(-(n // _NS) // 8) * 8) * _NS


def _seg_sum_kernel(n, e_pad, d):
    chunks = e_pad // _C
    per_tile = chunks // (_NC * _NS)
    rows_per_sub = _pad_rows(n) // _NS
    n_pad = rows_per_sub * _NS
    mesh = plsc.VectorSubcoreMesh(core_axis_name="c", subcore_axis_name="s")

    @functools.partial(
        pl.kernel,
        out_type=jax.ShapeDtypeStruct((_NC, n_pad, d), jnp.float32),
        mesh=mesh,
        compiler_params=pltpu.CompilerParams(use_tc_tiling_on_sc=False),
        scratch_types=[
            pltpu.VMEM_SHARED((n_pad, d), jnp.float32),
            pltpu.VMEM((_C,), jnp.int32),
            pltpu.VMEM((_C,), jnp.int32),
            pltpu.VMEM((_C, d), jnp.float32),
        ],
    )
    def seg_kernel(g_hbm, src_hbm, dst_hbm, zeros_hbm, out_hbm,
                   acc, srcb, dstb, buf):
        c = lax.axis_index("c")
        s = lax.axis_index("s")
        wid = c * _NS + s
        pltpu.sync_copy(zeros_hbm, acc.at[pl.ds(s * rows_per_sub, rows_per_sub)])
        plsc.subcore_barrier()
        base = wid * per_tile * _C

        @pl.loop(0, per_tile)
        def _(j):
            off = base + j * _C
            pltpu.sync_copy(src_hbm.at[pl.ds(off, _C)], srcb)
            pltpu.sync_copy(dst_hbm.at[pl.ds(off, _C)], dstb)
            pltpu.sync_copy(g_hbm.at[srcb], buf)
            pltpu.sync_copy(buf, acc.at[dstb], add=True)

        plsc.subcore_barrier()
        row = s * rows_per_sub
        pltpu.sync_copy(acc.at[pl.ds(row, rows_per_sub)],
                        out_hbm.at[c, pl.ds(row, rows_per_sub)])

    return seg_kernel


def _deg_kernel(n, e_pad):
    d = 8
    chunks = e_pad // _C
    per_tile = chunks // (_NC * _NS)
    rows_per_sub = _pad_rows(n) // _NS
    n_pad = rows_per_sub * _NS
    mesh = plsc.VectorSubcoreMesh(core_axis_name="c", subcore_axis_name="s")

    @functools.partial(
        pl.kernel,
        out_type=jax.ShapeDtypeStruct((_NC, n_pad, d), jnp.float32),
        mesh=mesh,
        compiler_params=pltpu.CompilerParams(use_tc_tiling_on_sc=False),
        scratch_types=[
            pltpu.VMEM_SHARED((n_pad, d), jnp.float32),
            pltpu.VMEM((_C,), jnp.int32),
            pltpu.VMEM((_C, d), jnp.float32),
        ],
    )
    def deg_kernel(dst_hbm, ones_hbm, zeros_hbm, out_hbm, acc, dstb, onesv):
        c = lax.axis_index("c")
        s = lax.axis_index("s")
        wid = c * _NS + s
        pltpu.sync_copy(zeros_hbm, acc.at[pl.ds(s * rows_per_sub, rows_per_sub)])
        pltpu.sync_copy(ones_hbm, onesv)
        plsc.subcore_barrier()
        base = wid * per_tile * _C

        @pl.loop(0, per_tile)
        def _(j):
            pltpu.sync_copy(dst_hbm.at[pl.ds(base + j * _C, _C)], dstb)
            pltpu.sync_copy(onesv, acc.at[dstb], add=True)

        plsc.subcore_barrier()
        row = s * rows_per_sub
        pltpu.sync_copy(acc.at[pl.ds(row, rows_per_sub)],
                        out_hbm.at[c, pl.ds(row, rows_per_sub)])

    return deg_kernel


def _tc_call(body, n, br, out_widths, ins, blocked):
    grid = (n // br,)
    in_specs = []
    for a, is_blocked in zip(ins, blocked):
        if is_blocked:
            in_specs.append(pl.BlockSpec((br,) + a.shape[1:],
                                         lambda i, nd=a.ndim: (i,) + (0,) * (nd - 1)))
        else:
            in_specs.append(pl.BlockSpec(a.shape, lambda i, nd=a.ndim: (0,) * nd))
    out_specs = [pl.BlockSpec((br, w), lambda i: (i, 0)) for w in out_widths]
    out_shape = [jax.ShapeDtypeStruct((n, w), jnp.float32) for w in out_widths]
    if len(out_widths) == 1:
        out_specs, out_shape = out_specs[0], out_shape[0]
    return pl.pallas_call(
        body, grid=grid, in_specs=in_specs, out_specs=out_specs,
        out_shape=out_shape)(*ins)


def kernel(x, edge_index, We1, be1, We2, be2, Wd1, bd1, Wd2, bd2):
    n = x.shape[0]
    e = edge_index.shape[1]
    quant = _C * _NC * _NS
    e_pad = -(-e // quant) * quant
    pad = e_pad - e
    src1d = jnp.concatenate([edge_index[0], jnp.zeros((pad,), jnp.int32)])
    dst1d = jnp.concatenate([edge_index[1], jnp.full((pad,), n, jnp.int32)])
    rows_per_sub = _pad_rows(n) // _NS
    z16 = jnp.zeros((rows_per_sub, 16), jnp.float32)
    z8 = jnp.zeros((rows_per_sub, 8), jnp.float32)
    ones_c = jnp.ones((_C, 8), jnp.float32)
    be1r = be1.reshape(1, 16)
    be2r = be2.reshape(1, 8)
    bd1r = bd1.reshape(1, 16)
    bd2r = bd2.reshape(1, 12)

    seg16 = _seg_sum_kernel(n, e_pad, 16)
    seg8 = _seg_sum_kernel(n, e_pad, 8)
    degk = _deg_kernel(n, e_pad)
    br = 10000

    degp = degk(dst1d, ones_c, z8)

    def tc0(dA, dB, xr, dis_o, g0_o):
        deg = dA[:, :1] + dB[:, :1] + 1.0
        dis = lax.rsqrt(deg)
        dis_o[...] = dis
        g0_o[...] = jnp.concatenate(
            [dis * xr[...], jnp.zeros((xr.shape[0], 4), jnp.float32)], axis=1)

    dis, g0 = _tc_call(tc0, n, br, [1, 16],
                       [degp[0], degp[1], x], [True, True, True])

    p = seg16(g0, src1d, dst1d, z16)

    def tc1(pA, pB, g0r, disr, W1, b1, W2, g2_o):
        dis_ = disr[...]
        u = (dis_ * (pA[...] + pB[...] + g0r[...]))[:, :12]
        a1 = jnp.maximum(jnp.dot(u, W1[...],
                                 preferred_element_type=jnp.float32) + b1[...], 0.0)
        g2_o[...] = dis_ * jnp.dot(a1, W2[...],
                                   preferred_element_type=jnp.float32)

    g2 = _tc_call(tc1, n, br, [8],
                  [p[0], p[1], g0, dis, We1, be1r, We2],
                  [True, True, True, True, False, False, False])

    p = seg8(g2, src1d, dst1d, z8)

    def tc2(pA, pB, g2r, disr, b2, g3_o):
        dis_ = disr[...]
        a2 = dis_ * (pA[...] + pB[...] + g2r[...]) + b2[...]
        g3_o[...] = dis_ * a2

    g3 = _tc_call(tc2, n, br, [8],
                  [p[0], p[1], g2, dis, be2r],
                  [True, True, True, True, False])

    p = seg8(g3, src1d, dst1d, z8)

    def tc3(pA, pB, g3r, disr, W1, b1, W2, g4_o):
        dis_ = disr[...]
        u = dis_ * (pA[...] + pB[...] + g3r[...])
        h3 = jnp.maximum(jnp.dot(u, W1[...],
                                 preferred_element_type=jnp.float32) + b1[...], 0.0)
        g4 = dis_ * jnp.dot(h3, W2[...], preferred_element_type=jnp.float32)
        g4_o[...] = jnp.concatenate(
            [g4, jnp.zeros((g4.shape[0], 4), jnp.float32)], axis=1)

    g4 = _tc_call(tc3, n, br, [16],
                  [p[0], p[1], g3, dis, Wd1, bd1r, Wd2],
                  [True, True, True, True, False, False, False])

    p = seg16(g4, src1d, dst1d, z16)

    def tc4(pA, pB, g4r, disr, b4, out_o):
        dis_ = disr[...]
        u = (dis_ * (pA[...] + pB[...] + g4r[...]))[:, :12]
        out_o[...] = jax.nn.sigmoid(u + b4[...])

    out = _tc_call(tc4, n, br, [12],
                   [p[0], p[1], g4, dis, bd2r],
                   [True, True, True, True, False])
    return out


# speedup vs baseline: 37.3014x; 2.4590x over previous
import functools

import jax
import jax.numpy as jnp
from jax import lax
from jax.experimental import pallas as pl
from jax.experimental.pallas import tpu as pltpu
from jax.experimental.pallas import tpu_sc as plsc

_C = 128
_NC = 2
_NS = 16


def _pad_rows(n):
    return (-(-(n // _NS) // 8) * 8) * _NS


def _seg_sum_kernel(n, e_pad, d):
    chunks = e_pad // _C
    per_tile = chunks // (_NC * _NS)
    assert per_tile % 4 == 0 and per_tile >= 8
    rows_per_sub = _pad_rows(n) // _NS
    n_pad = rows_per_sub * _NS
    mesh = plsc.VectorSubcoreMesh(core_axis_name="c", subcore_axis_name="s")

    @functools.partial(
        pl.kernel,
        out_type=jax.ShapeDtypeStruct((_NC, n_pad, d), jnp.float32),
        mesh=mesh,
        compiler_params=pltpu.CompilerParams(use_tc_tiling_on_sc=False),
        scratch_types=[
            pltpu.VMEM_SHARED((n_pad, d), jnp.float32),
            [pltpu.VMEM((_C,), jnp.int32) for _ in range(4)],
            [pltpu.VMEM((_C,), jnp.int32) for _ in range(4)],
            [pltpu.VMEM((_C, d), jnp.float32) for _ in range(2)],
            [pltpu.SemaphoreType.DMA for _ in range(4)],
            [pltpu.SemaphoreType.DMA for _ in range(2)],
            [pltpu.SemaphoreType.DMA for _ in range(2)],
        ],
    )
    def seg_kernel(g_hbm, src_hbm, dst_hbm, zeros_hbm, out_hbm,
                   acc, srcb, dstb, buf, isem, gsem, ssem):
        c = lax.axis_index("c")
        s = lax.axis_index("s")
        wid = c * _NS + s
        pltpu.sync_copy(zeros_hbm, acc.at[pl.ds(s * rows_per_sub, rows_per_sub)])
        plsc.subcore_barrier()
        base = wid * per_tile * _C

        def issue_i(t, k):
            off = base + t * _C
            pltpu.async_copy(src_hbm.at[pl.ds(off, _C)], srcb[k], isem[k])
            pltpu.async_copy(dst_hbm.at[pl.ds(off, _C)], dstb[k], isem[k])

        def wait_i(k):
            pltpu.make_async_copy(src_hbm.at[pl.ds(0, _C)], srcb[k], isem[k]).wait()
            pltpu.make_async_copy(dst_hbm.at[pl.ds(0, _C)], dstb[k], isem[k]).wait()

        def issue_g(k, b):
            pltpu.async_copy(g_hbm.at[srcb[k]], buf[b], gsem[b])

        def wait_g(b):
            pltpu.make_async_copy(g_hbm.at[srcb[0]], buf[b], gsem[b]).wait()

        def issue_s(b, k):
            pltpu.async_copy(buf[b], acc.at[dstb[k]], ssem[b], add=True)

        def wait_s(b):
            pltpu.make_async_copy(buf[b], acc.at[dstb[0]], ssem[b]).wait()

        def step(t, b2, k4, w_s, w_gprev, ahead):
            if w_s:
                wait_s(b2)
            wait_i(k4)
            issue_g(k4, b2)
            if w_gprev:
                wait_g(1 - b2)
                issue_s(1 - b2, (t + 3) % 4)
            if ahead:
                issue_i(t + 2, (t + 2) % 4 if isinstance(t, int) else None)

        issue_i(0, 0)
        issue_i(1, 1)
        step(0, 0, 0, w_s=False, w_gprev=False, ahead=True)
        step(1, 1, 1, w_s=False, w_gprev=True, ahead=True)

        @pl.loop(2, per_tile - 2, step=4)
        def _(t0):
            for u in range(4):
                b2, k4 = (2 + u) % 2, (2 + u) % 4
                t = t0 + u
                wait_s(b2)
                wait_i(k4)
                issue_g(k4, b2)
                wait_g(1 - b2)
                issue_s(1 - b2, (k4 + 3) % 4)
                issue_i(t + 2, (k4 + 2) % 4)

        for t in (per_tile - 2, per_tile - 1):
            b2, k4 = t % 2, t % 4
            wait_s(b2)
            wait_i(k4)
            issue_g(k4, b2)
            wait_g(1 - b2)
            issue_s(1 - b2, (k4 + 3) % 4)
        b2 = (per_tile - 1) % 2
        wait_g(b2)
        issue_s(b2, (per_tile - 1) % 4)
        wait_s(1 - b2)
        wait_s(b2)

        plsc.subcore_barrier()
        row = s * rows_per_sub
        pltpu.sync_copy(acc.at[pl.ds(row, rows_per_sub)],
                        out_hbm.at[c, pl.ds(row, rows_per_sub)])

    return seg_kernel


def _deg_kernel(n, e_pad):
    d = 8
    chunks = e_pad // _C
    per_tile = chunks // (_NC * _NS)
    rows_per_sub = _pad_rows(n) // _NS
    n_pad = rows_per_sub * _NS
    mesh = plsc.VectorSubcoreMesh(core_axis_name="c", subcore_axis_name="s")

    @functools.partial(
        pl.kernel,
        out_type=jax.ShapeDtypeStruct((_NC, n_pad, d), jnp.float32),
        mesh=mesh,
        compiler_params=pltpu.CompilerParams(use_tc_tiling_on_sc=False),
        scratch_types=[
            pltpu.VMEM_SHARED((n_pad, d), jnp.float32),
            [pltpu.VMEM((_C,), jnp.int32) for _ in range(4)],
            pltpu.VMEM((_C, d), jnp.float32),
            [pltpu.SemaphoreType.DMA for _ in range(4)],
            [pltpu.SemaphoreType.DMA for _ in range(4)],
        ],
    )
    def deg_kernel(dst_hbm, ones_hbm, zeros_hbm, out_hbm,
                   acc, dstb, onesv, isem, ssem):
        c = lax.axis_index("c")
        s = lax.axis_index("s")
        wid = c * _NS + s
        pltpu.sync_copy(zeros_hbm, acc.at[pl.ds(s * rows_per_sub, rows_per_sub)])
        pltpu.sync_copy(ones_hbm, onesv)
        plsc.subcore_barrier()
        base = wid * per_tile * _C

        def issue_i(t, k):
            pltpu.async_copy(dst_hbm.at[pl.ds(base + t * _C, _C)], dstb[k],
                             isem[k])

        def wait_i(k):
            pltpu.make_async_copy(dst_hbm.at[pl.ds(0, _C)], dstb[k],
                                  isem[k]).wait()

        def issue_s(k):
            pltpu.async_copy(onesv, acc.at[dstb[k]], ssem[k], add=True)

        def wait_s(k):
            pltpu.make_async_copy(onesv, acc.at[dstb[0]], ssem[k]).wait()

        issue_i(0, 0)
        issue_i(1, 1)
        wait_i(0); issue_s(0); issue_i(2, 2)
        wait_i(1); issue_s(1); issue_i(3, 3)

        @pl.loop(2, per_tile - 2, step=4)
        def _(t0):
            for u in range(4):
                k4 = (2 + u) % 4
                wait_i(k4)
                issue_s(k4)
                wait_s((k4 + 2) % 4)
                issue_i(t0 + u + 2, (k4 + 2) % 4)

        for t in (per_tile - 2, per_tile - 1):
            k4 = t % 4
            wait_i(k4)
            issue_s(k4)
        for k4 in range(4):
            wait_s(k4)

        plsc.subcore_barrier()
        row = s * rows_per_sub
        pltpu.sync_copy(acc.at[pl.ds(row, rows_per_sub)],
                        out_hbm.at[c, pl.ds(row, rows_per_sub)])

    return deg_kernel


def _tc_call(body, n, br, out_widths, ins, blocked):
    grid = (n // br,)
    in_specs = []
    for a, is_blocked in zip(ins, blocked):
        if is_blocked:
            in_specs.append(pl.BlockSpec((br,) + a.shape[1:],
                                         lambda i, nd=a.ndim: (i,) + (0,) * (nd - 1)))
        else:
            in_specs.append(pl.BlockSpec(a.shape, lambda i, nd=a.ndim: (0,) * nd))
    out_specs = [pl.BlockSpec((br, w), lambda i: (i, 0)) for w in out_widths]
    out_shape = [jax.ShapeDtypeStruct((n, w), jnp.float32) for w in out_widths]
    if len(out_widths) == 1:
        out_specs, out_shape = out_specs[0], out_shape[0]
    return pl.pallas_call(
        body, grid=grid, in_specs=in_specs, out_specs=out_specs,
        out_shape=out_shape)(*ins)


def kernel(x, edge_index, We1, be1, We2, be2, Wd1, bd1, Wd2, bd2):
    n = x.shape[0]
    e = edge_index.shape[1]
    quant = _C * _NC * _NS * 4
    e_pad = -(-e // quant) * quant
    pad = e_pad - e
    src1d = jnp.concatenate([edge_index[0], jnp.zeros((pad,), jnp.int32)])
    dst1d = jnp.concatenate([edge_index[1], jnp.full((pad,), n, jnp.int32)])
    rows_per_sub = _pad_rows(n) // _NS
    z16 = jnp.zeros((rows_per_sub, 16), jnp.float32)
    z8 = jnp.zeros((rows_per_sub, 8), jnp.float32)
    ones_c = jnp.ones((_C, 8), jnp.float32)
    be1r = be1.reshape(1, 16)
    be2r = be2.reshape(1, 8)
    bd1r = bd1.reshape(1, 16)
    bd2r = bd2.reshape(1, 12)

    seg16 = _seg_sum_kernel(n, e_pad, 16)
    seg8 = _seg_sum_kernel(n, e_pad, 8)
    degk = _deg_kernel(n, e_pad)
    br = 10000

    degp = degk(dst1d, ones_c, z8)

    def tc0(dA, dB, xr, dis_o, g0_o):
        deg = dA[:, :1] + dB[:, :1] + 1.0
        dis = lax.rsqrt(deg)
        dis_o[...] = dis
        g0_o[...] = jnp.concatenate(
            [dis * xr[...], jnp.zeros((xr.shape[0], 4), jnp.float32)], axis=1)

    dis, g0 = _tc_call(tc0, n, br, [1, 16],
                       [degp[0], degp[1], x], [True, True, True])

    p = seg16(g0, src1d, dst1d, z16)

    def tc1(pA, pB, g0r, disr, W1, b1, W2, g2_o):
        dis_ = disr[...]
        u = (dis_ * (pA[...] + pB[...] + g0r[...]))[:, :12]
        a1 = jnp.maximum(jnp.dot(u, W1[...],
                                 preferred_element_type=jnp.float32) + b1[...], 0.0)
        g2_o[...] = dis_ * jnp.dot(a1, W2[...],
                                   preferred_element_type=jnp.float32)

    g2 = _tc_call(tc1, n, br, [8],
                  [p[0], p[1], g0, dis, We1, be1r, We2],
                  [True, True, True, True, False, False, False])

    p = seg8(g2, src1d, dst1d, z8)

    def tc2(pA, pB, g2r, disr, b2, g3_o):
        dis_ = disr[...]
        a2 = dis_ * (pA[...] + pB[...] + g2r[...]) + b2[...]
        g3_o[...] = dis_ * a2

    g3 = _tc_call(tc2, n, br, [8],
                  [p[0], p[1], g2, dis, be2r],
                  [True, True, True, True, False])

    p = seg8(g3, src1d, dst1d, z8)

    def tc3(pA, pB, g3r, disr, W1, b1, W2, g4_o):
        dis_ = disr[...]
        u = dis_ * (pA[...] + pB[...] + g3r[...])
        h3 = jnp.maximum(jnp.dot(u, W1[...],
                                 preferred_element_type=jnp.float32) + b1[...], 0.0)
        g4 = dis_ * jnp.dot(h3, W2[...], preferred_element_type=jnp.float32)
        g4_o[...] = jnp.concatenate(
            [g4, jnp.zeros((g4.shape[0], 4), jnp.float32)], axis=1)

    g4 = _tc_call(tc3, n, br, [16],
                  [p[0], p[1], g3, dis, Wd1, bd1r, Wd2],
                  [True, True, True, True, False, False, False])

    p = seg16(g4, src1d, dst1d, z16)

    def tc4(pA, pB, g4r, disr, b4, out_o):
        dis_ = disr[...]
        u = (dis_ * (pA[...] + pB[...] + g4r[...]))[:, :12]
        out_o[...] = jax.nn.sigmoid(u + b4[...])

    out = _tc_call(tc4, n, br, [12],
                   [p[0], p[1], g4, dis, bd2r],
                   [True, True, True, True, False])
    return out


# speedup vs baseline: 41.9110x; 1.1236x over previous
import functools

import jax
import jax.numpy as jnp
from jax import lax
from jax.experimental import pallas as pl
from jax.experimental.pallas import tpu as pltpu
from jax.experimental.pallas import tpu_sc as plsc

_C = 128
_NC = 2
_NS = 16


def _pad_rows(n):
    return (-(-(n // _NS) // 8) * 8) * _NS


def _seg_sum_kernel(n, e, e_pad, d):
    chunks = e_pad // _C
    real_chunks = e // _C
    per_tile = chunks // (_NC * _NS)
    assert per_tile % 4 == 0 and per_tile >= 8
    rows_per_sub = _pad_rows(n) // _NS
    n_pad = rows_per_sub * _NS
    mesh = plsc.VectorSubcoreMesh(core_axis_name="c", subcore_axis_name="s")

    @functools.partial(
        pl.kernel,
        out_type=jax.ShapeDtypeStruct((_NC, n_pad, d), jnp.float32),
        mesh=mesh,
        compiler_params=pltpu.CompilerParams(use_tc_tiling_on_sc=False),
        scratch_types=[
            pltpu.VMEM_SHARED((n_pad, d), jnp.float32),
            [pltpu.VMEM((_C,), jnp.int32) for _ in range(4)],
            [pltpu.VMEM((_C,), jnp.int32) for _ in range(4)],
            [pltpu.VMEM((_C, d), jnp.float32) for _ in range(2)],
            [pltpu.SemaphoreType.DMA for _ in range(4)],
            [pltpu.SemaphoreType.DMA for _ in range(2)],
            [pltpu.SemaphoreType.DMA for _ in range(2)],
        ],
    )
    def seg_kernel(g_hbm, ei_hbm, dummy_hbm, zeros_hbm, out_hbm,
                   acc, srcb, dstb, buf, isem, gsem, ssem):
        c = lax.axis_index("c")
        s = lax.axis_index("s")
        wid = c * _NS + s
        pltpu.sync_copy(zeros_hbm, acc.at[pl.ds(s * rows_per_sub, rows_per_sub)])
        plsc.subcore_barrier()
        base = wid * per_tile * _C

        def issue_i(t, k):
            tg = base // _C + t

            @pl.when(tg < real_chunks)
            def _():
                off = tg * _C
                pltpu.async_copy(ei_hbm.at[0, pl.ds(off, _C)], srcb[k], isem[k])
                pltpu.async_copy(ei_hbm.at[1, pl.ds(off, _C)], dstb[k], isem[k])

            @pl.when(tg >= real_chunks)
            def _():
                off = (tg - real_chunks) * _C
                pltpu.async_copy(dummy_hbm.at[0, pl.ds(off, _C)], srcb[k], isem[k])
                pltpu.async_copy(dummy_hbm.at[1, pl.ds(off, _C)], dstb[k], isem[k])

        def wait_i(k):
            pltpu.make_async_copy(dummy_hbm.at[0, pl.ds(0, _C)], srcb[k], isem[k]).wait()
            pltpu.make_async_copy(dummy_hbm.at[1, pl.ds(0, _C)], dstb[k], isem[k]).wait()

        def issue_g(k, b):
            pltpu.async_copy(g_hbm.at[srcb[k]], buf[b], gsem[b])

        def wait_g(b):
            pltpu.make_async_copy(g_hbm.at[srcb[0]], buf[b], gsem[b]).wait()

        def issue_s(b, k):
            pltpu.async_copy(buf[b], acc.at[dstb[k]], ssem[b], add=True)

        def wait_s(b):
            pltpu.make_async_copy(buf[b], acc.at[dstb[0]], ssem[b]).wait()

        def step(t, b2, k4, w_s, w_gprev, ahead):
            if w_s:
                wait_s(b2)
            wait_i(k4)
            issue_g(k4, b2)
            if w_gprev:
                wait_g(1 - b2)
                issue_s(1 - b2, (t + 3) % 4)
            if ahead:
                issue_i(t + 2, (t + 2) % 4 if isinstance(t, int) else None)

        issue_i(0, 0)
        issue_i(1, 1)
        step(0, 0, 0, w_s=False, w_gprev=False, ahead=True)
        step(1, 1, 1, w_s=False, w_gprev=True, ahead=True)

        @pl.loop(2, per_tile - 2, step=4)
        def _(t0):
            for u in range(4):
                b2, k4 = (2 + u) % 2, (2 + u) % 4
                t = t0 + u
                wait_s(b2)
                wait_i(k4)
                issue_g(k4, b2)
                wait_g(1 - b2)
                issue_s(1 - b2, (k4 + 3) % 4)
                issue_i(t + 2, (k4 + 2) % 4)

        for t in (per_tile - 2, per_tile - 1):
            b2, k4 = t % 2, t % 4
            wait_s(b2)
            wait_i(k4)
            issue_g(k4, b2)
            wait_g(1 - b2)
            issue_s(1 - b2, (k4 + 3) % 4)
        b2 = (per_tile - 1) % 2
        wait_g(b2)
        issue_s(b2, (per_tile - 1) % 4)
        wait_s(1 - b2)
        wait_s(b2)

        plsc.subcore_barrier()
        row = s * rows_per_sub
        pltpu.sync_copy(acc.at[pl.ds(row, rows_per_sub)],
                        out_hbm.at[c, pl.ds(row, rows_per_sub)])

    return seg_kernel


def _deg_kernel(n, e, e_pad):
    d = 8
    chunks = e_pad // _C
    real_chunks = e // _C
    per_tile = chunks // (_NC * _NS)
    rows_per_sub = _pad_rows(n) // _NS
    n_pad = rows_per_sub * _NS
    mesh = plsc.VectorSubcoreMesh(core_axis_name="c", subcore_axis_name="s")

    @functools.partial(
        pl.kernel,
        out_type=jax.ShapeDtypeStruct((_NC, n_pad, d), jnp.float32),
        mesh=mesh,
        compiler_params=pltpu.CompilerParams(use_tc_tiling_on_sc=False),
        scratch_types=[
            pltpu.VMEM_SHARED((n_pad, d), jnp.float32),
            [pltpu.VMEM((_C,), jnp.int32) for _ in range(4)],
            pltpu.VMEM((_C, d), jnp.float32),
            [pltpu.SemaphoreType.DMA for _ in range(4)],
            [pltpu.SemaphoreType.DMA for _ in range(4)],
        ],
    )
    def deg_kernel(ei_hbm, dummy_hbm, ones_hbm, zeros_hbm, out_hbm,
                   acc, dstb, onesv, isem, ssem):
        c = lax.axis_index("c")
        s = lax.axis_index("s")
        wid = c * _NS + s
        pltpu.sync_copy(zeros_hbm, acc.at[pl.ds(s * rows_per_sub, rows_per_sub)])
        pltpu.sync_copy(ones_hbm, onesv)
        plsc.subcore_barrier()
        base = wid * per_tile * _C

        def issue_i(t, k):
            tg = base // _C + t

            @pl.when(tg < real_chunks)
            def _():
                pltpu.async_copy(ei_hbm.at[1, pl.ds(tg * _C, _C)], dstb[k],
                                 isem[k])

            @pl.when(tg >= real_chunks)
            def _():
                pltpu.async_copy(
                    dummy_hbm.at[1, pl.ds((tg - real_chunks) * _C, _C)],
                    dstb[k], isem[k])

        def wait_i(k):
            pltpu.make_async_copy(dummy_hbm.at[1, pl.ds(0, _C)], dstb[k],
                                  isem[k]).wait()

        def issue_s(k):
            pltpu.async_copy(onesv, acc.at[dstb[k]], ssem[k], add=True)

        def wait_s(k):
            pltpu.make_async_copy(onesv, acc.at[dstb[0]], ssem[k]).wait()

        issue_i(0, 0)
        issue_i(1, 1)
        wait_i(0); issue_s(0); issue_i(2, 2)
        wait_i(1); issue_s(1); issue_i(3, 3)

        @pl.loop(2, per_tile - 2, step=4)
        def _(t0):
            for u in range(4):
                k4 = (2 + u) % 4
                wait_i(k4)
                issue_s(k4)
                wait_s((k4 + 2) % 4)
                issue_i(t0 + u + 2, (k4 + 2) % 4)

        for t in (per_tile - 2, per_tile - 1):
            k4 = t % 4
            wait_i(k4)
            issue_s(k4)
        for k4 in range(4):
            wait_s(k4)

        plsc.subcore_barrier()
        row = s * rows_per_sub
        pltpu.sync_copy(acc.at[pl.ds(row, rows_per_sub)],
                        out_hbm.at[c, pl.ds(row, rows_per_sub)])

    return deg_kernel


def _tc_call(body, n, br, out_widths, ins):
    grid = (n // br,)
    in_specs = []
    arrs = []
    for a, mode in ins:
        arrs.append(a)
        if mode == 'b':
            in_specs.append(pl.BlockSpec((br,) + a.shape[1:],
                                         lambda i, nd=a.ndim: (i,) + (0,) * (nd - 1)))
        elif mode == 'p0':
            in_specs.append(pl.BlockSpec((1, br, a.shape[2]),
                                         lambda i: (0, i, 0)))
        elif mode == 'p1':
            in_specs.append(pl.BlockSpec((1, br, a.shape[2]),
                                         lambda i: (1, i, 0)))
        else:
            in_specs.append(pl.BlockSpec(a.shape, lambda i, nd=a.ndim: (0,) * nd))
    out_specs = [pl.BlockSpec((br, w), lambda i: (i, 0)) for w in out_widths]
    out_shape = [jax.ShapeDtypeStruct((n, w), jnp.float32) for w in out_widths]
    if len(out_widths) == 1:
        out_specs, out_shape = out_specs[0], out_shape[0]
    return pl.pallas_call(
        body, grid=grid, in_specs=in_specs, out_specs=out_specs,
        out_shape=out_shape)(*arrs)


def kernel(x, edge_index, We1, be1, We2, be2, Wd1, bd1, Wd2, bd2):
    n = x.shape[0]
    e = edge_index.shape[1]
    quant = _C * _NC * _NS * 4
    e_pad = -(-e // quant) * quant
    pad = e_pad - e
    dummy = jnp.concatenate([jnp.zeros((1, pad), jnp.int32),
                             jnp.full((1, pad), n, jnp.int32)])
    rows_per_sub = _pad_rows(n) // _NS
    z16 = jnp.zeros((rows_per_sub, 16), jnp.float32)
    z8 = jnp.zeros((rows_per_sub, 8), jnp.float32)
    ones_c = jnp.ones((_C, 8), jnp.float32)
    be1r = be1.reshape(1, 16)
    be2r = be2.reshape(1, 8)
    bd1r = bd1.reshape(1, 16)
    bd2r = bd2.reshape(1, 12)

    seg16 = _seg_sum_kernel(n, e, e_pad, 16)
    seg8 = _seg_sum_kernel(n, e, e_pad, 8)
    degk = _deg_kernel(n, e, e_pad)
    br = 10000

    degp = degk(edge_index, dummy, ones_c, z8)

    def tc0(dP0, dP1, xr, dis_o, g0_o):
        deg = dP0[0][:, :1] + dP1[0][:, :1] + 1.0
        dis = lax.rsqrt(deg)
        dis_o[...] = dis
        g0_o[...] = jnp.concatenate(
            [dis * xr[...], jnp.zeros((xr.shape[0], 4), jnp.float32)], axis=1)

    dis, g0 = _tc_call(tc0, n, br, [1, 16],
                       [(degp, 'p0'), (degp, 'p1'), (x, 'b')])

    p = seg16(g0, edge_index, dummy, z16)

    def tc1(pA, pB, g0r, disr, W1, b1, W2, g2_o):
        dis_ = disr[...]
        u = (dis_ * (pA[0] + pB[0] + g0r[...]))[:, :12]
        a1 = jnp.maximum(jnp.dot(u, W1[...],
                                 preferred_element_type=jnp.float32) + b1[...], 0.0)
        g2_o[...] = dis_ * jnp.dot(a1, W2[...],
                                   preferred_element_type=jnp.float32)

    g2 = _tc_call(tc1, n, br, [8],
                  [(p, 'p0'), (p, 'p1'), (g0, 'b'), (dis, 'b'),
                   (We1, 'w'), (be1r, 'w'), (We2, 'w')])

    p = seg8(g2, edge_index, dummy, z8)

    def tc2(pA, pB, g2r, disr, b2, g3_o):
        dis_ = disr[...]
        a2 = dis_ * (pA[0] + pB[0] + g2r[...]) + b2[...]
        g3_o[...] = dis_ * a2

    g3 = _tc_call(tc2, n, br, [8],
                  [(p, 'p0'), (p, 'p1'), (g2, 'b'), (dis, 'b'), (be2r, 'w')])

    p = seg8(g3, edge_index, dummy, z8)

    def tc3(pA, pB, g3r, disr, W1, b1, W2, g4_o):
        dis_ = disr[...]
        u = dis_ * (pA[0] + pB[0] + g3r[...])
        h3 = jnp.maximum(jnp.dot(u, W1[...],
                                 preferred_element_type=jnp.float32) + b1[...], 0.0)
        g4 = dis_ * jnp.dot(h3, W2[...], preferred_element_type=jnp.float32)
        g4_o[...] = jnp.concatenate(
            [g4, jnp.zeros((g4.shape[0], 4), jnp.float32)], axis=1)

    g4 = _tc_call(tc3, n, br, [16],
                  [(p, 'p0'), (p, 'p1'), (g3, 'b'), (dis, 'b'),
                   (Wd1, 'w'), (bd1r, 'w'), (Wd2, 'w')])

    p = seg16(g4, edge_index, dummy, z16)

    def tc4(pA, pB, g4r, disr, b4, out_o):
        dis_ = disr[...]
        u = (dis_ * (pA[0] + pB[0] + g4r[...]))[:, :12]
        out_o[...] = jax.nn.sigmoid(u + b4[...])

    out = _tc_call(tc4, n, br, [12],
                   [(p, 'p0'), (p, 'p1'), (g4, 'b'), (dis, 'b'), (bd2r, 'w')])
    return out


# speedup vs baseline: 43.0898x; 1.0281x over previous
import functools

import jax
import jax.numpy as jnp
from jax import lax
from jax.experimental import pallas as pl
from jax.experimental.pallas import tpu as pltpu
from jax.experimental.pallas import tpu_sc as plsc

_C = 128
_NC = 2
_NS = 16


def _pad_rows(n):
    return (-(-(n // _NS) // 8) * 8) * _NS


def _seg_sum_kernel(n, e, e_pad, d):
    chunks = e_pad // _C
    real_chunks = e // _C
    per_tile = chunks // (_NC * _NS)
    assert per_tile % 8 == 0 and per_tile >= 24
    rows_per_sub = _pad_rows(n) // _NS
    n_pad = rows_per_sub * _NS
    mesh = plsc.VectorSubcoreMesh(core_axis_name="c", subcore_axis_name="s")

    @functools.partial(
        pl.kernel,
        out_type=jax.ShapeDtypeStruct((_NC, n_pad, d), jnp.float32),
        mesh=mesh,
        compiler_params=pltpu.CompilerParams(use_tc_tiling_on_sc=False),
        scratch_types=[
            pltpu.VMEM_SHARED((n_pad, d), jnp.float32),
            [pltpu.VMEM((_C,), jnp.int32) for _ in range(8)],
            [pltpu.VMEM((_C,), jnp.int32) for _ in range(8)],
            [pltpu.VMEM((_C, d), jnp.float32) for _ in range(4)],
            [pltpu.SemaphoreType.DMA for _ in range(8)],
            [pltpu.SemaphoreType.DMA for _ in range(4)],
            [pltpu.SemaphoreType.DMA for _ in range(4)],
        ],
    )
    def seg_kernel(g_hbm, ei_hbm, dummy_hbm, zeros_hbm, out_hbm,
                   acc, srcb, dstb, buf, isem, gsem, ssem):
        c = lax.axis_index("c")
        s = lax.axis_index("s")
        wid = c * _NS + s
        pltpu.sync_copy(zeros_hbm, acc.at[pl.ds(s * rows_per_sub, rows_per_sub)])
        plsc.subcore_barrier()
        base = wid * per_tile

        def issue_i(t, k):
            tg = base + t

            @pl.when(tg < real_chunks)
            def _():
                off = tg * _C
                pltpu.async_copy(ei_hbm.at[0, pl.ds(off, _C)], srcb[k], isem[k])
                pltpu.async_copy(ei_hbm.at[1, pl.ds(off, _C)], dstb[k], isem[k])

            @pl.when(tg >= real_chunks)
            def _():
                off = (tg - real_chunks) * _C
                pltpu.async_copy(dummy_hbm.at[0, pl.ds(off, _C)], srcb[k], isem[k])
                pltpu.async_copy(dummy_hbm.at[1, pl.ds(off, _C)], dstb[k], isem[k])

        def wait_i(k):
            pltpu.make_async_copy(dummy_hbm.at[0, pl.ds(0, _C)], srcb[k], isem[k]).wait()
            pltpu.make_async_copy(dummy_hbm.at[1, pl.ds(0, _C)], dstb[k], isem[k]).wait()

        def issue_g(k8, b4):
            pltpu.async_copy(g_hbm.at[srcb[k8]], buf[b4], gsem[b4])

        def wait_g(b4):
            pltpu.make_async_copy(g_hbm.at[srcb[0]], buf[b4], gsem[b4]).wait()

        def issue_s(b4, k8):
            pltpu.async_copy(buf[b4], acc.at[dstb[k8]], ssem[b4], add=True)

        def wait_s(b4):
            pltpu.make_async_copy(buf[b4], acc.at[dstb[0]], ssem[b4]).wait()

        def step(tv, t, do_ws=True, do_wi=True, do_g=True, do_wg=True,
                 do_s=True, do_i=True):
            if do_ws:
                wait_s((t + 1) % 4)
            if do_wi:
                wait_i((t + 1) % 8)
            if do_g:
                issue_g((t + 1) % 8, (t + 1) % 4)
            if do_wg:
                wait_g(t % 4)
            if do_s:
                issue_s(t % 4, t % 8)
            if do_i:
                issue_i(tv + 3, (t + 3) % 8)

        issue_i(0, 0)
        issue_i(1, 1)
        issue_i(2, 2)
        wait_i(0)
        issue_g(0, 0)
        for t in range(0, 3):
            step(t, t, do_ws=False)
        for t in range(3, 8):
            step(t, t)

        @pl.loop(8, per_tile - 8, step=8)
        def _(t0):
            for u in range(8):
                step(t0 + u, u)

        for t in range(per_tile - 8, per_tile - 3):
            step(t, t % 8)
        step(per_tile - 3, (per_tile - 3) % 8, do_i=False)
        step(per_tile - 2, (per_tile - 2) % 8, do_i=False)
        step(per_tile - 1, (per_tile - 1) % 8, do_wi=False, do_g=False,
             do_i=False)
        for t in range(per_tile - 3, per_tile):
            wait_s(t % 4)

        plsc.subcore_barrier()
        row = s * rows_per_sub
        pltpu.sync_copy(acc.at[pl.ds(row, rows_per_sub)],
                        out_hbm.at[c, pl.ds(row, rows_per_sub)])

    return seg_kernel


def _deg_kernel(n, e, e_pad):
    d = 8
    chunks = e_pad // _C
    real_chunks = e // _C
    per_tile = chunks // (_NC * _NS)
    rows_per_sub = _pad_rows(n) // _NS
    n_pad = rows_per_sub * _NS
    mesh = plsc.VectorSubcoreMesh(core_axis_name="c", subcore_axis_name="s")

    @functools.partial(
        pl.kernel,
        out_type=jax.ShapeDtypeStruct((_NC, n_pad, d), jnp.float32),
        mesh=mesh,
        compiler_params=pltpu.CompilerParams(use_tc_tiling_on_sc=False),
        scratch_types=[
            pltpu.VMEM_SHARED((n_pad, d), jnp.float32),
            [pltpu.VMEM((_C,), jnp.int32) for _ in range(4)],
            pltpu.VMEM((_C, d), jnp.float32),
            [pltpu.SemaphoreType.DMA for _ in range(4)],
            [pltpu.SemaphoreType.DMA for _ in range(4)],
        ],
    )
    def deg_kernel(ei_hbm, dummy_hbm, ones_hbm, zeros_hbm, out_hbm,
                   acc, dstb, onesv, isem, ssem):
        c = lax.axis_index("c")
        s = lax.axis_index("s")
        wid = c * _NS + s
        pltpu.sync_copy(zeros_hbm, acc.at[pl.ds(s * rows_per_sub, rows_per_sub)])
        pltpu.sync_copy(ones_hbm, onesv)
        plsc.subcore_barrier()
        base = wid * per_tile * _C

        def issue_i(t, k):
            tg = base // _C + t

            @pl.when(tg < real_chunks)
            def _():
                pltpu.async_copy(ei_hbm.at[1, pl.ds(tg * _C, _C)], dstb[k],
                                 isem[k])

            @pl.when(tg >= real_chunks)
            def _():
                pltpu.async_copy(
                    dummy_hbm.at[1, pl.ds((tg - real_chunks) * _C, _C)],
                    dstb[k], isem[k])

        def wait_i(k):
            pltpu.make_async_copy(dummy_hbm.at[1, pl.ds(0, _C)], dstb[k],
                                  isem[k]).wait()

        def issue_s(k):
            pltpu.async_copy(onesv, acc.at[dstb[k]], ssem[k], add=True)

        def wait_s(k):
            pltpu.make_async_copy(onesv, acc.at[dstb[0]], ssem[k]).wait()

        issue_i(0, 0)
        issue_i(1, 1)
        wait_i(0); issue_s(0); issue_i(2, 2)
        wait_i(1); issue_s(1); issue_i(3, 3)

        @pl.loop(2, per_tile - 2, step=4)
        def _(t0):
            for u in range(4):
                k4 = (2 + u) % 4
                wait_i(k4)
                issue_s(k4)
                wait_s((k4 + 2) % 4)
                issue_i(t0 + u + 2, (k4 + 2) % 4)

        for t in (per_tile - 2, per_tile - 1):
            k4 = t % 4
            wait_i(k4)
            issue_s(k4)
        for k4 in range(4):
            wait_s(k4)

        plsc.subcore_barrier()
        row = s * rows_per_sub
        pltpu.sync_copy(acc.at[pl.ds(row, rows_per_sub)],
                        out_hbm.at[c, pl.ds(row, rows_per_sub)])

    return deg_kernel


def _tc_call(body, n, br, out_widths, ins):
    grid = (n // br,)
    in_specs = []
    arrs = []
    for a, mode in ins:
        arrs.append(a)
        if mode == 'b':
            in_specs.append(pl.BlockSpec((br,) + a.shape[1:],
                                         lambda i, nd=a.ndim: (i,) + (0,) * (nd - 1)))
        elif mode == 'p0':
            in_specs.append(pl.BlockSpec((1, br, a.shape[2]),
                                         lambda i: (0, i, 0)))
        elif mode == 'p1':
            in_specs.append(pl.BlockSpec((1, br, a.shape[2]),
                                         lambda i: (1, i, 0)))
        else:
            in_specs.append(pl.BlockSpec(a.shape, lambda i, nd=a.ndim: (0,) * nd))
    out_specs = [pl.BlockSpec((br, w), lambda i: (i, 0)) for w in out_widths]
    out_shape = [jax.ShapeDtypeStruct((n, w), jnp.float32) for w in out_widths]
    if len(out_widths) == 1:
        out_specs, out_shape = out_specs[0], out_shape[0]
    return pl.pallas_call(
        body, grid=grid, in_specs=in_specs, out_specs=out_specs,
        out_shape=out_shape)(*arrs)


def kernel(x, edge_index, We1, be1, We2, be2, Wd1, bd1, Wd2, bd2):
    n = x.shape[0]
    e = edge_index.shape[1]
    quant = _C * _NC * _NS * 8
    e_pad = -(-e // quant) * quant
    pad = e_pad - e
    dummy = jnp.concatenate([jnp.zeros((1, pad), jnp.int32),
                             jnp.full((1, pad), n, jnp.int32)])
    rows_per_sub = _pad_rows(n) // _NS
    z16 = jnp.zeros((rows_per_sub, 16), jnp.float32)
    z8 = jnp.zeros((rows_per_sub, 8), jnp.float32)
    ones_c = jnp.ones((_C, 8), jnp.float32)
    be1r = be1.reshape(1, 16)
    be2r = be2.reshape(1, 8)
    bd1r = bd1.reshape(1, 16)
    bd2r = bd2.reshape(1, 12)

    seg16 = _seg_sum_kernel(n, e, e_pad, 16)
    seg8 = _seg_sum_kernel(n, e, e_pad, 8)
    degk = _deg_kernel(n, e, e_pad)
    br = 10000

    degp = degk(edge_index, dummy, ones_c, z8)

    def tc0(dP0, dP1, xr, dis_o, g0_o):
        deg = dP0[0][:, :1] + dP1[0][:, :1] + 1.0
        dis = lax.rsqrt(deg)
        dis_o[...] = dis
        g0_o[...] = jnp.concatenate(
            [dis * xr[...], jnp.zeros((xr.shape[0], 4), jnp.float32)], axis=1)

    dis, g0 = _tc_call(tc0, n, br, [1, 16],
                       [(degp, 'p0'), (degp, 'p1'), (x, 'b')])

    p = seg16(g0, edge_index, dummy, z16)

    def tc1(pA, pB, g0r, disr, W1, b1, W2, g2_o):
        dis_ = disr[...]
        u = (dis_ * (pA[0] + pB[0] + g0r[...]))[:, :12]
        a1 = jnp.maximum(jnp.dot(u, W1[...],
                                 preferred_element_type=jnp.float32) + b1[...], 0.0)
        g2_o[...] = dis_ * jnp.dot(a1, W2[...],
                                   preferred_element_type=jnp.float32)

    g2 = _tc_call(tc1, n, br, [8],
                  [(p, 'p0'), (p, 'p1'), (g0, 'b'), (dis, 'b'),
                   (We1, 'w'), (be1r, 'w'), (We2, 'w')])

    p = seg8(g2, edge_index, dummy, z8)

    def tc2(pA, pB, g2r, disr, b2, g3_o):
        dis_ = disr[...]
        a2 = dis_ * (pA[0] + pB[0] + g2r[...]) + b2[...]
        g3_o[...] = dis_ * a2

    g3 = _tc_call(tc2, n, br, [8],
                  [(p, 'p0'), (p, 'p1'), (g2, 'b'), (dis, 'b'), (be2r, 'w')])

    p = seg8(g3, edge_index, dummy, z8)

    def tc3(pA, pB, g3r, disr, W1, b1, W2, g4_o):
        dis_ = disr[...]
        u = dis_ * (pA[0] + pB[0] + g3r[...])
        h3 = jnp.maximum(jnp.dot(u, W1[...],
                                 preferred_element_type=jnp.float32) + b1[...], 0.0)
        g4 = dis_ * jnp.dot(h3, W2[...], preferred_element_type=jnp.float32)
        g4_o[...] = jnp.concatenate(
            [g4, jnp.zeros((g4.shape[0], 4), jnp.float32)], axis=1)

    g4 = _tc_call(tc3, n, br, [16],
                  [(p, 'p0'), (p, 'p1'), (g3, 'b'), (dis, 'b'),
                   (Wd1, 'w'), (bd1r, 'w'), (Wd2, 'w')])

    p = seg16(g4, edge_index, dummy, z16)

    def tc4(pA, pB, g4r, disr, b4, out_o):
        dis_ = disr[...]
        u = (dis_ * (pA[0] + pB[0] + g4r[...]))[:, :12]
        out_o[...] = jax.nn.sigmoid(u + b4[...])

    out = _tc_call(tc4, n, br, [12],
                   [(p, 'p0'), (p, 'p1'), (g4, 'b'), (dis, 'b'), (bd2r, 'w')])
    return out


# speedup vs baseline: 55.0564x; 1.2777x over previous
import functools

import jax
import jax.numpy as jnp
from jax import lax
from jax.experimental import pallas as pl
from jax.experimental.pallas import tpu as pltpu
from jax.experimental.pallas import tpu_sc as plsc

_C = 128
_NC = 2
_NS = 16


def _pad_rows(n):
    return (-(-(n // _NS) // 8) * 8) * _NS


def _seg_sum_kernel(n, e, e_pad, d):
    B = 4 * _C
    supers = e_pad // B
    real_supers = e // B
    per_super = supers // (_NC * _NS)
    assert per_super % 4 == 0 and per_super >= 12
    assert e % B == 0
    rows_per_sub = _pad_rows(n) // _NS
    n_pad = rows_per_sub * _NS
    mesh = plsc.VectorSubcoreMesh(core_axis_name="c", subcore_axis_name="s")

    @functools.partial(
        pl.kernel,
        out_type=jax.ShapeDtypeStruct((_NC, n_pad, d), jnp.float32),
        mesh=mesh,
        compiler_params=pltpu.CompilerParams(use_tc_tiling_on_sc=False),
        scratch_types=[
            pltpu.VMEM_SHARED((n_pad, d), jnp.float32),
            [pltpu.VMEM((B,), jnp.int32) for _ in range(4)],
            [pltpu.VMEM((_C,), jnp.int32) for _ in range(16)],
            [pltpu.VMEM((B, d), jnp.float32) for _ in range(2)],
            [pltpu.SemaphoreType.DMA for _ in range(4)],
            [pltpu.SemaphoreType.DMA for _ in range(16)],
            [pltpu.SemaphoreType.DMA for _ in range(2)],
            [pltpu.SemaphoreType.DMA for _ in range(2)],
        ],
    )
    def seg_kernel(g_hbm, ei_hbm, dummy_hbm, zeros_hbm, out_hbm,
                   acc, srcb, dstb, buf, isem, idsem, gsem, ssem):
        c = lax.axis_index("c")
        s = lax.axis_index("s")
        wid = c * _NS + s
        pltpu.sync_copy(zeros_hbm, acc.at[pl.ds(s * rows_per_sub, rows_per_sub)])
        plsc.subcore_barrier()
        base = wid * per_super

        def issue_isrc(t, k):
            tg = base + t

            @pl.when(tg < real_supers)
            def _():
                pltpu.async_copy(ei_hbm.at[0, pl.ds(tg * B, B)], srcb[k], isem[k])

            @pl.when(tg >= real_supers)
            def _():
                pltpu.async_copy(dummy_hbm.at[0, pl.ds((tg - real_supers) * B, B)],
                                 srcb[k], isem[k])

        def wait_isrc(k):
            pltpu.make_async_copy(dummy_hbm.at[0, pl.ds(0, B)], srcb[k],
                                  isem[k]).wait()

        def issue_idst(t, m4):
            tg = base + t

            @pl.when(tg < real_supers)
            def _():
                for q in range(4):
                    pltpu.async_copy(ei_hbm.at[1, pl.ds(tg * B + q * _C, _C)],
                                     dstb[m4 * 4 + q], idsem[m4 * 4 + q])

            @pl.when(tg >= real_supers)
            def _():
                for q in range(4):
                    pltpu.async_copy(
                        dummy_hbm.at[1, pl.ds((tg - real_supers) * B + q * _C, _C)],
                        dstb[m4 * 4 + q], idsem[m4 * 4 + q])

        def wait_idst(slot):
            pltpu.make_async_copy(dummy_hbm.at[1, pl.ds(0, _C)], dstb[slot],
                                  idsem[slot]).wait()

        def issue_g(k4, b2):
            pltpu.async_copy(g_hbm.at[srcb[k4]], buf[b2], gsem[b2])

        def wait_g(b2):
            pltpu.make_async_copy(g_hbm.at[srcb[0]], buf[b2], gsem[b2]).wait()

        def issue_s(b2, m4, q):
            pltpu.async_copy(buf[b2].at[pl.ds(q * _C, _C)],
                             acc.at[dstb[m4 * 4 + q]], ssem[b2], add=True)

        def wait_s4(b2):
            for _ in range(4):
                pltpu.make_async_copy(buf[b2].at[pl.ds(0, _C)],
                                      acc.at[dstb[0]], ssem[b2]).wait()

        def step(tv, t, do_ws, do_g, do_isrc, do_idst):
            b2 = t % 2
            if do_g:
                if do_ws:
                    wait_s4((t + 1) % 2)
                wait_isrc((t + 1) % 4)
                issue_g((t + 1) % 4, (t + 1) % 2)
            wait_g(b2)
            for q in range(4):
                wait_idst((t % 4) * 4 + q)
                issue_s(b2, t % 4, q)
            if do_isrc:
                issue_isrc(tv + 3, (t + 3) % 4)
            if do_idst:
                issue_idst(tv + 1, (t + 1) % 4)

        issue_isrc(0, 0)
        issue_isrc(1, 1)
        issue_isrc(2, 2)
        issue_idst(0, 0)
        wait_isrc(0)
        issue_g(0, 0)
        step(0, 0, do_ws=False, do_g=True, do_isrc=True, do_idst=True)
        for t in range(1, 4):
            step(t, t, do_ws=True, do_g=True, do_isrc=True, do_idst=True)

        @pl.loop(4, per_super - 4, step=4)
        def _(t0):
            for u in range(4):
                step(t0 + u, u, do_ws=True, do_g=True, do_isrc=True,
                     do_idst=True)

        pt = per_super
        step(pt - 4, 0, do_ws=True, do_g=True, do_isrc=True, do_idst=True)
        step(pt - 3, 1, do_ws=True, do_g=True, do_isrc=False, do_idst=True)
        step(pt - 2, 2, do_ws=True, do_g=True, do_isrc=False, do_idst=True)
        step(pt - 1, 3, do_ws=False, do_g=False, do_isrc=False, do_idst=False)
        wait_s4((pt - 2) % 2)
        wait_s4((pt - 1) % 2)

        plsc.subcore_barrier()
        row = s * rows_per_sub
        pltpu.sync_copy(acc.at[pl.ds(row, rows_per_sub)],
                        out_hbm.at[c, pl.ds(row, rows_per_sub)])

    return seg_kernel


def _deg_kernel(n, e, e_pad):
    d = 8
    chunks = e_pad // _C
    real_chunks = e // _C
    per_tile = chunks // (_NC * _NS)
    rows_per_sub = _pad_rows(n) // _NS
    n_pad = rows_per_sub * _NS
    mesh = plsc.VectorSubcoreMesh(core_axis_name="c", subcore_axis_name="s")

    @functools.partial(
        pl.kernel,
        out_type=jax.ShapeDtypeStruct((_NC, n_pad, d), jnp.float32),
        mesh=mesh,
        compiler_params=pltpu.CompilerParams(use_tc_tiling_on_sc=False),
        scratch_types=[
            pltpu.VMEM_SHARED((n_pad, d), jnp.float32),
            [pltpu.VMEM((_C,), jnp.int32) for _ in range(4)],
            pltpu.VMEM((_C, d), jnp.float32),
            [pltpu.SemaphoreType.DMA for _ in range(4)],
            [pltpu.SemaphoreType.DMA for _ in range(4)],
        ],
    )
    def deg_kernel(ei_hbm, dummy_hbm, ones_hbm, zeros_hbm, out_hbm,
                   acc, dstb, onesv, isem, ssem):
        c = lax.axis_index("c")
        s = lax.axis_index("s")
        wid = c * _NS + s
        pltpu.sync_copy(zeros_hbm, acc.at[pl.ds(s * rows_per_sub, rows_per_sub)])
        pltpu.sync_copy(ones_hbm, onesv)
        plsc.subcore_barrier()
        base = wid * per_tile * _C

        def issue_i(t, k):
            tg = base // _C + t

            @pl.when(tg < real_chunks)
            def _():
                pltpu.async_copy(ei_hbm.at[1, pl.ds(tg * _C, _C)], dstb[k],
                                 isem[k])

            @pl.when(tg >= real_chunks)
            def _():
                pltpu.async_copy(
                    dummy_hbm.at[1, pl.ds((tg - real_chunks) * _C, _C)],
                    dstb[k], isem[k])

        def wait_i(k):
            pltpu.make_async_copy(dummy_hbm.at[1, pl.ds(0, _C)], dstb[k],
                                  isem[k]).wait()

        def issue_s(k):
            pltpu.async_copy(onesv, acc.at[dstb[k]], ssem[k], add=True)

        def wait_s(k):
            pltpu.make_async_copy(onesv, acc.at[dstb[0]], ssem[k]).wait()

        issue_i(0, 0)
        issue_i(1, 1)
        wait_i(0); issue_s(0); issue_i(2, 2)
        wait_i(1); issue_s(1); issue_i(3, 3)

        @pl.loop(2, per_tile - 2, step=4)
        def _(t0):
            for u in range(4):
                k4 = (2 + u) % 4
                wait_i(k4)
                issue_s(k4)
                wait_s((k4 + 2) % 4)
                issue_i(t0 + u + 2, (k4 + 2) % 4)

        for t in (per_tile - 2, per_tile - 1):
            k4 = t % 4
            wait_i(k4)
            issue_s(k4)
        for k4 in range(4):
            wait_s(k4)

        plsc.subcore_barrier()
        row = s * rows_per_sub
        pltpu.sync_copy(acc.at[pl.ds(row, rows_per_sub)],
                        out_hbm.at[c, pl.ds(row, rows_per_sub)])

    return deg_kernel


def _tc_call(body, n, br, out_widths, ins):
    grid = (n // br,)
    in_specs = []
    arrs = []
    for a, mode in ins:
        arrs.append(a)
        if mode == 'b':
            in_specs.append(pl.BlockSpec((br,) + a.shape[1:],
                                         lambda i, nd=a.ndim: (i,) + (0,) * (nd - 1)))
        elif mode == 'p0':
            in_specs.append(pl.BlockSpec((1, br, a.shape[2]),
                                         lambda i: (0, i, 0)))
        elif mode == 'p1':
            in_specs.append(pl.BlockSpec((1, br, a.shape[2]),
                                         lambda i: (1, i, 0)))
        else:
            in_specs.append(pl.BlockSpec(a.shape, lambda i, nd=a.ndim: (0,) * nd))
    out_specs = [pl.BlockSpec((br, w), lambda i: (i, 0)) for w in out_widths]
    out_shape = [jax.ShapeDtypeStruct((n, w), jnp.float32) for w in out_widths]
    if len(out_widths) == 1:
        out_specs, out_shape = out_specs[0], out_shape[0]
    return pl.pallas_call(
        body, grid=grid, in_specs=in_specs, out_specs=out_specs,
        out_shape=out_shape)(*arrs)


def kernel(x, edge_index, We1, be1, We2, be2, Wd1, bd1, Wd2, bd2):
    n = x.shape[0]
    e = edge_index.shape[1]
    quant = _C * _NC * _NS * 8
    e_pad = -(-e // quant) * quant
    pad = e_pad - e
    dummy = jnp.concatenate([jnp.zeros((1, pad), jnp.int32),
                             jnp.full((1, pad), n, jnp.int32)])
    rows_per_sub = _pad_rows(n) // _NS
    z16 = jnp.zeros((rows_per_sub, 16), jnp.float32)
    z8 = jnp.zeros((rows_per_sub, 8), jnp.float32)
    ones_c = jnp.ones((_C, 8), jnp.float32)
    be1r = be1.reshape(1, 16)
    be2r = be2.reshape(1, 8)
    bd1r = bd1.reshape(1, 16)
    bd2r = bd2.reshape(1, 12)

    seg16 = _seg_sum_kernel(n, e, e_pad, 16)
    seg8 = _seg_sum_kernel(n, e, e_pad, 8)
    degk = _deg_kernel(n, e, e_pad)
    br = 10000

    degp = degk(edge_index, dummy, ones_c, z8)

    def tc0(dP0, dP1, xr, dis_o, g0_o):
        deg = dP0[0][:, :1] + dP1[0][:, :1] + 1.0
        dis = lax.rsqrt(deg)
        dis_o[...] = dis
        g0_o[...] = jnp.concatenate(
            [dis * xr[...], jnp.zeros((xr.shape[0], 4), jnp.float32)], axis=1)

    dis, g0 = _tc_call(tc0, n, br, [1, 16],
                       [(degp, 'p0'), (degp, 'p1'), (x, 'b')])

    p = seg16(g0, edge_index, dummy, z16)

    def tc1(pA, pB, g0r, disr, W1, b1, W2, g2_o):
        dis_ = disr[...]
        u = (dis_ * (pA[0] + pB[0] + g0r[...]))[:, :12]
        a1 = jnp.maximum(jnp.dot(u, W1[...],
                                 preferred_element_type=jnp.float32) + b1[...], 0.0)
        g2_o[...] = dis_ * jnp.dot(a1, W2[...],
                                   preferred_element_type=jnp.float32)

    g2 = _tc_call(tc1, n, br, [8],
                  [(p, 'p0'), (p, 'p1'), (g0, 'b'), (dis, 'b'),
                   (We1, 'w'), (be1r, 'w'), (We2, 'w')])

    p = seg8(g2, edge_index, dummy, z8)

    def tc2(pA, pB, g2r, disr, b2, g3_o):
        dis_ = disr[...]
        a2 = dis_ * (pA[0] + pB[0] + g2r[...]) + b2[...]
        g3_o[...] = dis_ * a2

    g3 = _tc_call(tc2, n, br, [8],
                  [(p, 'p0'), (p, 'p1'), (g2, 'b'), (dis, 'b'), (be2r, 'w')])

    p = seg8(g3, edge_index, dummy, z8)

    def tc3(pA, pB, g3r, disr, W1, b1, W2, g4_o):
        dis_ = disr[...]
        u = dis_ * (pA[0] + pB[0] + g3r[...])
        h3 = jnp.maximum(jnp.dot(u, W1[...],
                                 preferred_element_type=jnp.float32) + b1[...], 0.0)
        g4 = dis_ * jnp.dot(h3, W2[...], preferred_element_type=jnp.float32)
        g4_o[...] = jnp.concatenate(
            [g4, jnp.zeros((g4.shape[0], 4), jnp.float32)], axis=1)

    g4 = _tc_call(tc3, n, br, [16],
                  [(p, 'p0'), (p, 'p1'), (g3, 'b'), (dis, 'b'),
                   (Wd1, 'w'), (bd1r, 'w'), (Wd2, 'w')])

    p = seg16(g4, edge_index, dummy, z16)

    def tc4(pA, pB, g4r, disr, b4, out_o):
        dis_ = disr[...]
        u = (dis_ * (pA[0] + pB[0] + g4r[...]))[:, :12]
        out_o[...] = jax.nn.sigmoid(u + b4[...])

    out = _tc_call(tc4, n, br, [12],
                   [(p, 'p0'), (p, 'p1'), (g4, 'b'), (dis, 'b'), (bd2r, 'w')])
    return out
